# baseline (speedup 1.0000x reference)
import jax
import jax.numpy as jnp
from jax import lax
from jax.experimental import pallas as pl
from jax.experimental.pallas import tpu as pltpu
from jax.experimental.pallas import tpu_sc as plsc

N_FEATURES = 26
INPUT_DIM = 33
OUT_DIM = 32
BATCH = 16384

NC, NS, L = 2, 16, 16
NW = NC * NS
TOTAL = BATCH * N_FEATURES
PER_W = TOTAL // NW
G = 128
N_GROUPS = PER_W // G
CHUNK = 1024
NG = CHUNK // G
N_CHUNKS = PER_W // CHUNK
S_GROUPS = 4
OFF_LEN = 208
TAB_WORDS = N_FEATURES * INPUT_DIM * OUT_DIM


def _embed_body(idx_hbm, off_hbm, tab_hbm, out_hbm,
                idx_v, off_v, sp_tab, tab_v, rows0, rows1,
                sg0, sg1, sw0, sw1):
    cid = lax.axis_index("c")
    sid = lax.axis_index("s")
    wid = sid * NC + cid
    wbase = wid * PER_W

    @pl.when(sid == 0)
    def _stage():
        pltpu.sync_copy(tab_hbm, sp_tab)

    pltpu.sync_copy(tab_hbm, tab_v)
    pltpu.sync_copy(off_hbm, off_v)
    pltpu.sync_copy(idx_hbm.at[pl.ds(wbase // G, N_GROUPS)], idx_v)

    @plsc.parallel_loop(0, PER_W // L)
    def _precompute(i):
        r = i // (G // L)
        col = (i % (G // L)) * L
        off = off_v[pl.ds((i % (OFF_LEN // L)) * L, L)]
        idx_v[r, pl.ds(col, L)] = idx_v[r, pl.ds(col, L)] + off

    plsc.subcore_barrier()

    def vector_part(c, buf):
        @plsc.parallel_loop(0, (NG - S_GROUPS) * (G // L), unroll=2)
        def _vec(t):
            g = S_GROUPS + t // (G // L)
            v = t % (G // L)
            a16 = idx_v[c * NG + g, pl.ds(v * L, L)]
            for k in range(L):
                a = a16[k]
                row = g * G + v * L + k
                buf[row, pl.ds(0, L)] = tab_v[a, pl.ds(0, L)]
                buf[row, pl.ds(L, L)] = tab_v[a, pl.ds(L, L)]

    bufs = (rows0, rows1)
    gsems = (sg0, sg1)
    wsems = (sw0, sw1)
    pend_g = [None, None]
    pend_w = [None, None]

    for c in range(N_CHUNKS + 1):
        if c < N_CHUNKS:
            b = c % 2
            if pend_w[b] is not None:
                pend_w[b].wait()
            gs = []
            for g in range(S_GROUPS):
                cp = pltpu.make_async_copy(
                    sp_tab.at[idx_v.at[c * NG + g]],
                    bufs[b].at[pl.ds(g * G, G)],
                    gsems[b],
                )
                cp.start()
                gs.append(cp)
            pend_g[b] = gs
            vector_part(c, bufs[b])
        if c >= 1:
            b2 = (c - 1) % 2
            for cp in pend_g[b2]:
                cp.wait()
            wr = pltpu.make_async_copy(
                bufs[b2],
                out_hbm.at[pl.ds(wbase + (c - 1) * CHUNK, CHUNK)],
                wsems[b2],
            )
            wr.start()
            pend_w[b2] = wr

    pend_w[(N_CHUNKS - 1) % 2].wait()


def kernel(inputs, tables):
    idx_flat = inputs.reshape(TOTAL // G, G)
    tab_flat = tables.reshape(N_FEATURES * INPUT_DIM, OUT_DIM)
    off = jnp.tile(
        jnp.arange(N_FEATURES, dtype=jnp.int32) * INPUT_DIM,
        OFF_LEN // N_FEATURES,
    )

    run = pl.kernel(
        _embed_body,
        out_type=jax.ShapeDtypeStruct((TOTAL, OUT_DIM), jnp.float32),
        mesh=plsc.VectorSubcoreMesh(core_axis_name="c", subcore_axis_name="s"),
        scratch_types=[
            pltpu.VMEM((N_GROUPS, G), jnp.int32),
            pltpu.VMEM((OFF_LEN,), jnp.int32),
            pltpu.VMEM_SHARED((N_FEATURES * INPUT_DIM, OUT_DIM), jnp.float32),
            pltpu.VMEM((N_FEATURES * INPUT_DIM, OUT_DIM), jnp.float32),
            pltpu.VMEM((CHUNK, OUT_DIM), jnp.float32),
            pltpu.VMEM((CHUNK, OUT_DIM), jnp.float32),
            pltpu.SemaphoreType.DMA,
            pltpu.SemaphoreType.DMA,
            pltpu.SemaphoreType.DMA,
            pltpu.SemaphoreType.DMA,
        ],
        compiler_params=pltpu.CompilerParams(
            use_tc_tiling_on_sc=False,
            needs_layout_passes=False,
            disable_bounds_checks=True,
        ),
    )
    out = run(idx_flat, off, tab_flat)
    return out.reshape(BATCH, N_FEATURES * OUT_DIM)
